# baseline (speedup 1.0000x reference)
import jax
import jax.numpy as jnp
from jax import lax
from jax.experimental import pallas as pl
from jax.experimental.pallas import tpu as pltpu
from jax.experimental.pallas import tpu_sc as plsc

_LANES = 16
_CM = 256
_PAD = 1


def _build_sc_gather(nb, n, c, m):
    info = plsc.get_sparse_core_info()
    num_workers = info.num_cores * info.num_subcores
    assert num_workers % nb == 0
    parts = num_workers // nb
    assert m % parts == 0
    mper = m // parts
    assert mper % _CM == 0
    nch = mper // _CM
    assert nch % 2 == 0 and nch >= 4
    assert c % _LANES == 0
    ncb = c // _LANES

    def body(ft_hbm, idx_hbm, out_hbm, idx_v, g_a, g_b, t_a, t_b,
             s0, s1, t0, t1):
        gbufs = (g_a, g_b)
        tbufs = (t_a, t_b)
        insems = (s0, s1)
        osems = (t0, t1)
        w = lax.axis_index("s") * info.num_cores + lax.axis_index("c")
        b = w // parts
        mb0 = (w % parts) * mper
        pltpu.sync_copy(idx_hbm.at[b].at[pl.ds(mb0, mper)], idx_v)
        table = ft_hbm.at[b]
        lanes = lax.iota(jnp.int32, _LANES)

        def in_start(ch, k):
            pltpu.async_copy(
                table.at[idx_v.at[pl.ds(ch * _CM, _CM)]],
                gbufs[k], insems[k])

        def in_wait(k):
            pltpu.make_async_copy(
                table.at[idx_v.at[pl.ds(0, _CM)]],
                gbufs[k], insems[k]).wait()

        def t_start(ch, cb, p):
            pltpu.async_copy(
                tbufs[p].at[pl.ds(0, _LANES), pl.ds(0, _CM)],
                out_hbm.at[pl.ds(b * c + cb * _LANES, _LANES),
                           pl.ds(mb0 + ch * _CM, _CM)],
                osems[p])

        def t_wait(p):
            pltpu.make_async_copy(
                tbufs[p].at[pl.ds(0, _LANES), pl.ds(0, _CM)],
                out_hbm.at[pl.ds(0, _LANES), pl.ds(0, _CM)],
                osems[p]).wait()

        in_start(0, 0)
        in_start(1, 1)

        def grp(g, carry):
            for k in range(2):
                ch = g * 2 + k
                in_wait(k)
                for cb in range(ncb):
                    p = cb % 2
                    if k == 0 and cb < 2:
                        @pl.when(g > 0)
                        def _():
                            t_wait(p)
                    else:
                        t_wait(p)

                    @plsc.parallel_loop(0, _CM, unroll=8)
                    def _(mm):
                        vals = gbufs[k][mm, pl.ds(cb * _LANES, _LANES)]
                        plsc.store_scatter(
                            tbufs[p], [lanes, jnp.full((_LANES,), mm,
                                                       jnp.int32)], vals)

                    t_start(ch, cb, p)

                @pl.when(ch + 2 < nch)
                def _():
                    in_start(ch + 2, k)
            return carry

        lax.fori_loop(0, nch // 2, grp, 0)
        t_wait(0)
        t_wait(1)

    return pl.kernel(
        body,
        out_type=jax.ShapeDtypeStruct((nb * c, m), jnp.float32),
        mesh=plsc.VectorSubcoreMesh(core_axis_name="c", subcore_axis_name="s"),
        scratch_types=[
            pltpu.VMEM((mper,), jnp.int32),
            pltpu.VMEM((_CM, c), jnp.float32),
            pltpu.VMEM((_CM, c), jnp.float32),
            pltpu.VMEM((_LANES, _CM + _PAD), jnp.float32),
            pltpu.VMEM((_LANES, _CM + _PAD), jnp.float32),
            pltpu.SemaphoreType.DMA,
            pltpu.SemaphoreType.DMA,
            pltpu.SemaphoreType.DMA,
            pltpu.SemaphoreType.DMA,
        ],
        compiler_params=pltpu.CompilerParams(needs_layout_passes=False),
    )


def kernel(features, idx):
    nb, c, n = features.shape
    m = idx.shape[1]
    ft = jnp.swapaxes(features, 1, 2)
    idx32 = idx.astype(jnp.int32)
    gather = _build_sc_gather(nb, n, c, m)
    out2 = gather(ft, idx32)
    return out2.reshape(nb, c, m)
